# baseline (speedup 1.0000x reference)
import functools

import jax
import jax.numpy as jnp
from jax import lax
from jax.experimental import pallas as pl
from jax.experimental.pallas import tpu as pltpu
from jax.experimental.pallas import tpu_sc as plsc

WORD_DIM = 128
SMALL_DIM = 16
OUT_DIM = WORD_DIM + 3 * SMALL_DIM
PAD_DIM = 2 * WORD_DIM
L_ROW = 200
IDX_SPLIT = (128, 72)


@functools.partial(jax.jit, static_argnames=("batch",))
def _embed(wi, ti, p1i, p2i, word_W, tagp, p1p, p2p, *, batch):
    info = plsc.get_sparse_core_info()
    nc, ns = info.num_cores, info.num_subcores
    nw = nc * ns
    bpw = batch // nw
    assert bpw * nw == batch and bpw % 2 == 0
    half = bpw // 2

    mesh = plsc.VectorSubcoreMesh(core_axis_name="c", subcore_axis_name="s")

    per_set = (
        [pltpu.VMEM((L_ROW,), jnp.int32)] * 4
        + [
            pltpu.VMEM((L_ROW, WORD_DIM), jnp.float32),
            pltpu.VMEM((L_ROW, WORD_DIM), jnp.float32),
        ]
        + [pltpu.SemaphoreType.DMA] * 4
    )

    @functools.partial(
        pl.kernel,
        mesh=mesh,
        compiler_params=pltpu.CompilerParams(use_tc_tiling_on_sc=True),
        out_type=jax.ShapeDtypeStruct((batch, L_ROW, PAD_DIM), jnp.float32),
        scratch_types=per_set + per_set,
    )
    def embed_kernel(wi_h, ti_h, p1_h, p2_h, wW_h, tW_h, p1W_h, p2W_h,
                     out_h, *scratch):
        wid = lax.axis_index("s") * nc + lax.axis_index("c")

        n_per = len(per_set)
        sets = []
        for b in (0, 1):
            sc = scratch[b * n_per:(b + 1) * n_per]
            sets.append(dict(
                idx=sc[0:4], wbuf=sc[4], sbuf=sc[5],
                isem=sc[6], tsem=sc[7], gsem=sc[8], wsem=sc[9],
            ))
        idx_srcs = (wi_h, ti_h, p1_h, p2_h)

        def idx_copies(s, i, mk):
            st = sets[s]
            base = (wid * bpw + i) * L_ROW
            return [mk(idx_srcs[t].at[pl.ds(base, L_ROW)], st["idx"][t],
                       st["isem"])
                    for t in range(4)]

        def tag_copies(s, mk):
            st = sets[s]
            out = []
            off = 0
            for ln in IDX_SPLIT:
                out.append(mk(wW_h.at[st["idx"][0].at[pl.ds(off, ln)]],
                              st["wbuf"].at[pl.ds(off, ln)], st["tsem"]))
                out.append(mk(tW_h.at[st["idx"][1].at[pl.ds(off, ln)]],
                              st["sbuf"].at[pl.ds(off, ln)], st["tsem"]))
                off += ln
            return out

        def pos_copies(s):
            st = sets[s]
            for tbl, t in ((p1W_h, 2), (p2W_h, 3)):
                off = 0
                for ln in IDX_SPLIT:
                    pltpu.async_copy(
                        tbl.at[st["idx"][t].at[pl.ds(off, ln)]],
                        st["sbuf"].at[pl.ds(off, ln)], st["gsem"], add=True)
                    off += ln

        def pos_waits(s):
            st = sets[s]
            for tbl, t in ((p1W_h, 2), (p2W_h, 3)):
                off = 0
                for ln in IDX_SPLIT:
                    pltpu.make_async_copy(
                        tbl.at[st["idx"][t].at[pl.ds(off, ln)]],
                        st["sbuf"].at[pl.ds(off, ln)], st["gsem"]).wait()
                    off += ln

        def write_copies(s, i, mk):
            st = sets[s]
            row = wid * bpw + i
            return [
                mk(st["wbuf"], out_h.at[row, :, pl.ds(0, WORD_DIM)],
                   st["wsem"]),
                mk(st["sbuf"], out_h.at[row, :, pl.ds(WORD_DIM, WORD_DIM)],
                   st["wsem"]),
            ]

        fire = pltpu.async_copy

        def drain(copier, *args):
            for c in copier(*args, pltpu.make_async_copy):
                c.wait()

        def finish_gathers(s):
            drain(tag_copies, s)
            pos_copies(s)
            pos_waits(s)

        idx_copies(0, 0, fire)
        drain(idx_copies, 0, 0)
        tag_copies(0, fire)
        idx_copies(1, 1, fire)
        drain(idx_copies, 1, 1)
        tag_copies(1, fire)
        finish_gathers(0)
        write_copies(0, 0, fire)
        idx_copies(0, 2, fire)

        def pair(h, _):
            e = 2 * h
            o = e + 1
            drain(idx_copies, 0, e)
            drain(write_copies, 0, e - 2)
            tag_copies(0, fire)
            finish_gathers(1)
            write_copies(1, o - 2, fire)
            idx_copies(1, o, fire)
            drain(idx_copies, 1, o)
            drain(write_copies, 1, o - 2)
            tag_copies(1, fire)
            finish_gathers(0)
            write_copies(0, e, fire)

            @pl.when(h < half - 1)
            def _():
                idx_copies(0, o + 1, fire)

            return 0

        lax.fori_loop(1, half, pair, 0)

        finish_gathers(1)
        write_copies(1, bpw - 1, fire)
        drain(write_copies, 0, bpw - 2)
        drain(write_copies, 1, bpw - 1)

    return embed_kernel(wi, ti, p1i, p2i, word_W, tagp, p1p, p2p)


def kernel(word, tag, pos1, pos2, word_W, tag_W, pos1_W, pos2_W):
    B, L = word.shape
    n = B * L
    tagp = jnp.pad(tag_W, ((0, 0), (0, WORD_DIM - SMALL_DIM)))
    p1p = jnp.pad(pos1_W, ((0, 0), (SMALL_DIM, WORD_DIM - 2 * SMALL_DIM)))
    p2p = jnp.pad(pos2_W, ((0, 0), (2 * SMALL_DIM, WORD_DIM - 3 * SMALL_DIM)))
    out = _embed(
        word.reshape(n).astype(jnp.int32),
        tag.reshape(n).astype(jnp.int32),
        pos1.reshape(n).astype(jnp.int32),
        pos2.reshape(n).astype(jnp.int32),
        word_W, tagp, p1p, p2p,
        batch=B,
    )
    return out[..., :OUT_DIM]


# speedup vs baseline: 1.1213x; 1.1213x over previous
import functools

import jax
import jax.numpy as jnp
from jax import lax
from jax.experimental import pallas as pl
from jax.experimental.pallas import tpu as pltpu
from jax.experimental.pallas import tpu_sc as plsc

WORD_DIM = 128
SMALL_DIM = 16
OUT_DIM = WORD_DIM + 3 * SMALL_DIM
L_ROW = 200
TOK_STEP = 320
IDX_SPLIT = (128, 128, 64)


@functools.partial(jax.jit, static_argnames=("batch",))
def _embed(wi, ti, p1i, p2i, word_W, tag_W, pos1_W, pos2_W, *, batch):
    info = plsc.get_sparse_core_info()
    nc, ns = info.num_cores, info.num_subcores
    nw = nc * ns
    bpw = batch // nw
    spw = bpw * L_ROW // TOK_STEP
    assert bpw * nw == batch and spw % 2 == 0
    half = spw // 2

    mesh = plsc.VectorSubcoreMesh(core_axis_name="c", subcore_axis_name="s")

    per_set = (
        [pltpu.VMEM((TOK_STEP,), jnp.int32)] * 4
        + [
            pltpu.VMEM((TOK_STEP, WORD_DIM), jnp.float32),
            pltpu.VMEM((TOK_STEP, SMALL_DIM), jnp.float32),
            pltpu.VMEM((TOK_STEP, SMALL_DIM), jnp.float32),
            pltpu.VMEM((TOK_STEP, SMALL_DIM), jnp.float32),
        ]
        + [pltpu.SemaphoreType.DMA] * 3
    )

    @functools.partial(
        pl.kernel,
        mesh=mesh,
        compiler_params=pltpu.CompilerParams(use_tc_tiling_on_sc=False),
        out_type=jax.ShapeDtypeStruct((batch * L_ROW, OUT_DIM), jnp.float32),
        scratch_types=per_set + per_set,
    )
    def embed_kernel(wi_h, ti_h, p1_h, p2_h, wW_h, tW_h, p1W_h, p2W_h,
                     out_h, *scratch):
        wid = lax.axis_index("s") * nc + lax.axis_index("c")

        n_per = len(per_set)
        sets = []
        for b in (0, 1):
            sc = scratch[b * n_per:(b + 1) * n_per]
            sets.append(dict(
                idx=sc[0:4], data=sc[4:8], isem=sc[8], gsem=sc[9], wsem=sc[10],
            ))
        idx_srcs = (wi_h, ti_h, p1_h, p2_h)
        tables = (wW_h, tW_h, p1W_h, p2W_h)
        col_off = (0, WORD_DIM, WORD_DIM + SMALL_DIM, WORD_DIM + 2 * SMALL_DIM)
        col_w = (WORD_DIM, SMALL_DIM, SMALL_DIM, SMALL_DIM)

        def idx_copies(s, i, mk):
            st = sets[s]
            base = wid * bpw * L_ROW + i * TOK_STEP
            return [mk(idx_srcs[t].at[pl.ds(base, TOK_STEP)], st["idx"][t],
                       st["isem"])
                    for t in range(4)]

        def gather_copies(s, mk):
            st = sets[s]
            out = []
            for t in range(4):
                off = 0
                for ln in IDX_SPLIT:
                    out.append(mk(tables[t].at[st["idx"][t].at[pl.ds(off, ln)]],
                                  st["data"][t].at[pl.ds(off, ln)],
                                  st["gsem"]))
                    off += ln
            return out

        def write_copies(s, i, mk):
            st = sets[s]
            base = wid * bpw * L_ROW + i * TOK_STEP
            return [mk(st["data"][t],
                       out_h.at[pl.ds(base, TOK_STEP), pl.ds(col_off[t], col_w[t])],
                       st["wsem"])
                    for t in range(4)]

        fire = pltpu.async_copy

        def drain(copier, *args):
            for c in copier(*args, pltpu.make_async_copy):
                c.wait()

        idx_copies(0, 0, fire)
        drain(idx_copies, 0, 0)
        gather_copies(0, fire)
        idx_copies(1, 1, fire)
        drain(idx_copies, 1, 1)
        gather_copies(1, fire)
        drain(gather_copies, 0)
        write_copies(0, 0, fire)
        idx_copies(0, 2, fire)

        def pair(h, _):
            e = 2 * h
            o = e + 1
            drain(idx_copies, 0, e)
            drain(write_copies, 0, e - 2)
            gather_copies(0, fire)
            drain(gather_copies, 1)
            write_copies(1, o - 2, fire)
            idx_copies(1, o, fire)
            drain(idx_copies, 1, o)
            drain(write_copies, 1, o - 2)
            gather_copies(1, fire)
            drain(gather_copies, 0)
            write_copies(0, e, fire)

            @pl.when(h < half - 1)
            def _():
                idx_copies(0, o + 1, fire)

            return 0

        lax.fori_loop(1, half, pair, 0)

        drain(gather_copies, 1)
        write_copies(1, spw - 1, fire)
        drain(write_copies, 0, spw - 2)
        drain(write_copies, 1, spw - 1)

    return embed_kernel(wi, ti, p1i, p2i, word_W, tag_W, pos1_W, pos2_W)


def kernel(word, tag, pos1, pos2, word_W, tag_W, pos1_W, pos2_W):
    B, L = word.shape
    n = B * L
    return _embed(
        word.reshape(n).astype(jnp.int32),
        tag.reshape(n).astype(jnp.int32),
        pos1.reshape(n).astype(jnp.int32),
        pos2.reshape(n).astype(jnp.int32),
        word_W, tag_W, pos1_W, pos2_W,
        batch=B,
    ).reshape(B, L, OUT_DIM)
